# baseline (speedup 1.0000x reference)
import functools

import jax
import jax.numpy as jnp
from jax import lax
from jax.experimental import pallas as pl
from jax.experimental.pallas import tpu as pltpu
from jax.experimental.pallas import tpu_sc as plsc

B = 16
MAX_LEN = 2048
D = 1024
TOTAL = B * MAX_LEN // 2
HALF = MAX_LEN // 2

NC = 2
NS = 16
RPW = TOTAL // (NC * NS)

C = 16
NBUF = 6
DLY = 3
NCH = RPW // C
NZBUF = 4
PCH = -(-HALF // C)

_mesh = plsc.VectorSubcoreMesh(core_axis_name="c", subcore_axis_name="s")


@functools.partial(
    pl.kernel,
    mesh=_mesh,
    out_type=[
        jax.ShapeDtypeStruct((B * MAX_LEN, D), jnp.float32),
        jax.ShapeDtypeStruct((B * MAX_LEN,), jnp.int32),
    ],
    scratch_types=(
        [pltpu.VMEM((32,), jnp.int32),
         pltpu.VMEM((HALF,), jnp.int32),
         pltpu.VMEM((16,), jnp.int32)]
        + [pltpu.VMEM((C, D), jnp.float32) for _ in range(NBUF)]
        + [pltpu.VMEM((C,), jnp.int32) for _ in range(NBUF)]
        + [pltpu.SemaphoreType.DMA for _ in range(2 * NBUF + 3)]
    ),
    compiler_params=pltpu.CompilerParams(needs_layout_passes=False),
)
def _pack(cu_hbm, flat_hbm, zeros_hbm, padded_hbm, mask_hbm,
          cu_v, mask_v, pidx, *rest):
    bufs = rest[:NBUF]
    didx = rest[NBUF:2 * NBUF]
    insem = rest[2 * NBUF:3 * NBUF]
    outsem = rest[3 * NBUF:4 * NBUF]
    psem = rest[4 * NBUF:]

    wid = lax.axis_index("c") * NS + lax.axis_index("s")
    b = wid // 2
    p0 = (wid % 2) * HALF

    pltpu.sync_copy(cu_hbm, cu_v)
    lane = lax.iota(jnp.int32, 16)
    starts_vec = cu_v[pl.ds(0, 16)]
    ends_vec = starts_vec + cu_v[pl.ds(16, 16)]
    sel = lane == b
    len_b = jnp.sum(jnp.where(sel, cu_v[pl.ds(16, 16)], 0))
    ends = [jnp.sum(jnp.where(lane == j, ends_vec, 0)) for j in range(B)]

    n_real = jnp.clip(len_b - p0, 0, HALF)
    n_pad = HALF - n_real
    out0 = b * MAX_LEN + p0
    zbase = out0 + n_real

    fbase = wid * RPW

    def fill_didx(i, slot):
        for h in range(0, C, 16):
            t = fbase + i * C + h + lane
            bt = jnp.zeros((16,), jnp.int32)
            for j in range(B):
                bt = bt + (t >= ends[j]).astype(jnp.int32)
            s_bt = plsc.load_gather(cu_v, [bt])
            didx[slot][pl.ds(h, 16)] = bt * MAX_LEN + t - s_bt

    for j in range(NBUF):
        pltpu.make_async_copy(flat_hbm.at[pl.ds(fbase + j * C, C)],
                              bufs[j], insem[j]).start()

    for i in range(NCH):
        slot = i % NBUF
        g = i + NBUF - DLY
        if i - DLY >= 0 and g < NCH:
            gslot = g % NBUF
            pltpu.make_async_copy(bufs[gslot], padded_hbm.at[didx[gslot]],
                                  outsem[gslot]).wait()
            pltpu.make_async_copy(flat_hbm.at[pl.ds(fbase + g * C, C)],
                                  bufs[gslot], insem[gslot]).start()
        pltpu.make_async_copy(flat_hbm.at[pl.ds(fbase + i * C, C)],
                              bufs[slot], insem[slot]).wait()
        fill_didx(i, slot)
        pltpu.make_async_copy(bufs[slot], padded_hbm.at[didx[slot]],
                              outsem[slot]).start()

    for s in range(NBUF):
        pltpu.make_async_copy(bufs[s], padded_hbm.at[didx[s]],
                              outsem[s]).wait()

    end = out0 + HALF
    a0 = pl.multiple_of(zbase + (8 - (zbase % 8)) % 8, 8)
    nlin = end - a0
    nzch = nlin // C

    @pl.when(n_pad > 0)
    def _pads():
        pltpu.sync_copy(zeros_hbm, bufs[0])
        q = jnp.minimum(lane, n_pad - 1)
        pidx[pl.ds(0, 16)] = zbase + q
        pltpu.make_async_copy(bufs[0].at[pl.ds(0, 16)],
                              padded_hbm.at[pidx], psem[0]).start()

    for j in range(PCH):
        @pl.when(j < nzch)
        def _pad(j=j):
            if j >= NZBUF:
                pltpu.make_async_copy(bufs[0],
                                      padded_hbm.at[pl.ds(a0, C)],
                                      psem[1]).wait()
            pltpu.make_async_copy(bufs[0],
                                  padded_hbm.at[pl.ds(end - (j + 1) * C, C)],
                                  psem[1]).start()

    rem_base = a0
    for s in (8,):
        @pl.when(((nlin % C) & s) != 0)
        def _prem(s=s, rem_base=rem_base):
            pltpu.make_async_copy(
                bufs[0].at[pl.ds(0, s)],
                padded_hbm.at[pl.ds(pl.multiple_of(rem_base, 8), s)],
                psem[2]).start()

        rem_base = rem_base + jnp.where(((nlin % C) & s) != 0, s, 0)

    def mrow(k, carry):
        mask_v[pl.ds(k * 16, 16)] = (lane + (p0 + k * 16) < len_b).astype(
            jnp.int32)
        return carry

    lax.fori_loop(0, HALF // 16, mrow, 0)
    pltpu.sync_copy(mask_v, mask_hbm.at[pl.ds(out0, HALF)])

    @pl.when(n_pad > 0)
    def _pdrain0():
        pltpu.make_async_copy(bufs[0].at[pl.ds(0, 16)],
                              padded_hbm.at[pidx], psem[0]).wait()

    def _pdrain(j, carry):
        pltpu.make_async_copy(bufs[0], padded_hbm.at[pl.ds(a0, C)],
                              psem[1]).wait()
        return carry

    lax.fori_loop(0, jnp.minimum(nzch, NZBUF), _pdrain, 0)
    for s in (8,):
        @pl.when(((nlin % C) & s) != 0)
        def _premw(s=s):
            pltpu.make_async_copy(bufs[0].at[pl.ds(0, s)],
                                  padded_hbm.at[pl.ds(a0, s)],
                                  psem[2]).wait()


def kernel(flat, cu_seqlens):
    cu32 = jnp.concatenate([cu_seqlens[:B],
                            cu_seqlens[1:] - cu_seqlens[:-1]])
    zeros = jnp.zeros((C, D), jnp.float32)
    padded_flat, mask_flat = _pack(cu32, flat, zeros)
    return (padded_flat.reshape(B, MAX_LEN, D),
            mask_flat.reshape(B, MAX_LEN))
